# baseline (speedup 1.0000x reference)
import functools
import math

import jax
import jax.numpy as jnp
from jax import lax
from jax.experimental import pallas as pl
from jax.experimental.pallas import tpu as pltpu
from jax.experimental.pallas import tpu_sc as plsc

N = 5000
NUM_CLASSES = 91
C_PAD = 96
P_SUB = 8
P_LANE = 640
P_PAD = P_SUB * P_LANE
SCORE_THRESH = 0.05
NMS_THRESH = 0.5
DETS_PER_IMG = 100
IMG_H = 800.0
IMG_W = 800.0
BBOX_CLIP = math.log(1000.0 / 16.0)
NEG = -1.0
CAP = 163840
BIG = 2 ** 30


def _decode_body(logits_ref, dx_ref, dy_ref, dw_ref, dh_ref, prop_ref,
                 s_ref, x1_ref, y1_ref, x2_ref, y2_ref, cnt_ref, max_ref):
    px1 = prop_ref[0]
    py1 = prop_ref[1]
    px2 = prop_ref[2]
    py2 = prop_ref[3]
    w = px2 - px1
    h = py2 - py1
    cx = px1 + 0.5 * w
    cy = py1 + 0.5 * h

    logits = logits_ref[...]
    lmax = jnp.max(logits[:NUM_CLASSES], axis=0)
    e = jnp.exp(logits - lmax[None])
    denom = jnp.sum(e[:NUM_CLASSES], axis=0)
    scores = e / denom[None]

    dx = dx_ref[...] * (1.0 / 10.0)
    dy = dy_ref[...] * (1.0 / 10.0)
    dw = jnp.minimum(dw_ref[...] * (1.0 / 5.0), BBOX_CLIP)
    dh = jnp.minimum(dh_ref[...] * (1.0 / 5.0), BBOX_CLIP)
    pcx = dx * w[None] + cx[None]
    pcy = dy * h[None] + cy[None]
    pw = jnp.exp(dw) * w[None]
    ph = jnp.exp(dh) * h[None]
    x1 = jnp.clip(pcx - 0.5 * pw, 0.0, IMG_W)
    y1 = jnp.clip(pcy - 0.5 * ph, 0.0, IMG_H)
    x2 = jnp.clip(pcx + 0.5 * pw, 0.0, IMG_W)
    y2 = jnp.clip(pcy + 0.5 * ph, 0.0, IMG_H)

    pidx = (lax.broadcasted_iota(jnp.int32, (P_SUB, P_LANE), 0) * P_LANE
            + lax.broadcasted_iota(jnp.int32, (P_SUB, P_LANE), 1))
    crow = lax.broadcasted_iota(jnp.int32, (C_PAD, P_SUB, P_LANE), 0)
    valid = ((scores > SCORE_THRESH)
             & ((x2 - x1) >= 0.01)
             & ((y2 - y1) >= 0.01)
             & (pidx[None] < N)
             & (crow >= 1) & (crow < NUM_CLASSES))
    s = jnp.where(valid, scores, NEG)
    s_ref[...] = s
    x1_ref[...] = x1
    y1_ref[...] = y1
    x2_ref[...] = x2
    y2_ref[...] = y2

    cnts = jnp.sum(jnp.sum(jnp.where(valid, 1.0, 0.0), axis=2), axis=1)
    rowmax = jnp.max(jnp.max(s, axis=2), axis=1)
    cnt_row = jnp.concatenate(
        [cnts, jnp.zeros((128 - C_PAD,), jnp.float32)], axis=0).reshape(1, 128)
    max_row = jnp.concatenate(
        [rowmax, jnp.full((128 - C_PAD,), NEG, jnp.float32)],
        axis=0).reshape(1, 128)
    cnt_ref[...] = jnp.concatenate(
        [cnt_row, jnp.zeros((7, 128), jnp.float32)], axis=0)
    max_ref[...] = jnp.concatenate(
        [max_row, jnp.full((7, 128), NEG, jnp.float32)], axis=0)


def _sc_nms_body(s_hbm, x1_hbm, y1_hbm, x2_hbm, y2_hbm, cnt_hbm, max_hbm,
                 out_hbm,
                 row0, row1, row2, row3, row4,
                 cs0, cs1, cs2, cs3, cs4,
                 out_buf, cntf_loc, maxf_loc,
                 spm_s, spm_x1, spm_y1, spm_x2, spm_y2,
                 sem):
    f32 = jnp.float32
    i32 = jnp.int32
    cid = lax.axis_index("c")
    sid = lax.axis_index("s")
    iota = lax.iota(i32, 16)
    neg1 = jnp.full((16,), NEG, f32)
    zero16f = jnp.zeros((16,), f32)
    spms = (spm_s, spm_x1, spm_y1, spm_x2, spm_y2)
    hbms = (s_hbm, x1_hbm, y1_hbm, x2_hbm, y2_hbm)
    rows = (row0, row1, row2, row3, row4)
    css = (cs0, cs1, cs2, cs3, cs4)
    NVR = P_PAD // 16

    @pl.when(cid == 0)
    def _core0():
        pltpu.sync_copy(cnt_hbm.at[pl.ds(0, 128)], cntf_loc)
        pltpu.sync_copy(max_hbm.at[pl.ds(0, 128)], maxf_loc)
        ncnt_j = []
        cnt16_j = []
        off_j = []
        carry = jnp.int32(0)
        for j in range(6):
            n = cntf_loc[pl.ds(j * 16, 16)].astype(i32)
            n = jnp.minimum(jnp.maximum(n, 0), N)
            n16 = (n + 15) & -16
            capj = (n16 + 511) & -512
            cums = plsc.cumsum(capj)
            off_j.append(carry + cums - capj)
            carry = carry + jnp.max(cums)
            ncnt_j.append(n)
            cnt16_j.append(n16)

        for k in range(6):
            n_c = jnp.max(jnp.where(iota == sid, ncnt_j[k], 0))
            n16c = jnp.max(jnp.where(iota == sid, cnt16_j[k], 0))
            off_c = pl.multiple_of(jnp.minimum(
                jnp.max(jnp.where(iota == sid, off_j[k], 0)),
                CAP - P_PAD), 512)

            @pl.when(n16c > 0)
            def _compact(k=k, n_c=n_c, n16c=n16c, off_c=off_c):
                c = k * 16 + sid
                hs = [pltpu.async_copy(hbms[p].at[c], rows[p], sem)
                      for p in range(5)]
                for hnd in hs:
                    hnd.wait()

                def kbody(i, nloc):
                    sv = row0[pl.ds(i * 16, 16)]
                    mask = sv > 0.0

                    @pl.when(jnp.any(mask))
                    def _scat():
                        ps = plsc.cumsum(jnp.where(mask, 1, 0))
                        pos = nloc + ps - 1
                        for p in range(5):
                            v = rows[p][pl.ds(i * 16, 16)]
                            plsc.store_scatter(css[p], [pos], v, mask=mask)

                    return nloc + plsc.all_reduce_population_count(mask)

                lax.fori_loop(0, NVR, kbody, jnp.zeros((16,), i32))
                padidx = n_c + iota
                plsc.store_scatter(cs0, [padidx], neg1, mask=padidx < n16c)
                nch = (n16c + 511) >> 9

                def dbody(kk, _):
                    for p in range(5):
                        pltpu.sync_copy(
                            css[p].at[pl.ds(kk * 512, 512)],
                            spms[p].at[pl.ds(
                                pl.multiple_of(off_c + kk * 512, 512), 512)])
                    return 0

                lax.fori_loop(0, nch, dbody, 0)

        plsc.subcore_barrier()

        @pl.when(sid == 0)
        def _tile0():
            def zbody(i, _):
                out_buf[pl.ds(i * 16, 16)] = zero16f
                return 0
            lax.fori_loop(0, 52, zbody, 0)

            m_j = [maxf_loc[pl.ds(j * 16, 16)] for j in range(6)]
            enc_j = [off_j[j] * 8192 + cnt16_j[j] for j in range(6)]

            def it(t, m_state):
                vm = m_state[0]
                for j in range(1, 6):
                    vm = jnp.maximum(vm, m_state[j])
                vstar = jnp.max(vm)
                cond = vstar > 0.0
                csel = jnp.full((16,), BIG, i32)
                encv = jnp.zeros((16,), i32)
                for j in range(6):
                    hit = m_state[j] == vstar
                    csel = jnp.minimum(csel,
                                       jnp.where(hit, iota + 16 * j, BIG))
                cstar = jnp.min(csel)
                for j in range(6):
                    at_c = (iota + 16 * j) == cstar
                    encv = encv + jnp.where(at_c, enc_j[j], 0)
                enc = jnp.max(encv)
                n16 = jnp.minimum(enc & 8191, P_PAD)
                off = pl.multiple_of(
                    jnp.minimum(enc >> 13, CAP - P_PAD), 512)
                nch = (n16 + 511) >> 9
                nv = n16 >> 4

                def fetch(kk, _):
                    hs = [pltpu.async_copy(
                        spms[p].at[pl.ds(pl.multiple_of(off + kk * 512, 512),
                                         512)],
                        rows[p].at[pl.ds(kk * 512, 512)], sem)
                        for p in range(5)]
                    for hnd in hs:
                        hnd.wait()
                    return 0

                lax.fori_loop(0, nch, fetch, 0)

                def abody(i, best):
                    sv = row0[pl.ds(i * 16, 16)]
                    return jnp.minimum(
                        best, jnp.where(sv == vstar, i * 16 + iota, BIG))

                bestv = lax.fori_loop(0, nv, abody,
                                      jnp.full((16,), BIG, i32))
                irow = jnp.minimum(jnp.min(bestv), P_PAD - 16)

                irow_splat = jnp.full((16,), irow, i32)
                bx1 = plsc.load_gather(row1, [irow_splat])
                by1 = plsc.load_gather(row2, [irow_splat])
                bx2 = plsc.load_gather(row3, [irow_splat])
                by2 = plsc.load_gather(row4, [irow_splat])
                barea = (bx2 - bx1) * (by2 - by1)

                def ibody(i, runmax):
                    sl = pl.ds(i * 16, 16)
                    sv = row0[sl]
                    x1v = row1[sl]
                    y1v = row2[sl]
                    x2v = row3[sl]
                    y2v = row4[sl]
                    xx1 = jnp.maximum(bx1, x1v)
                    yy1 = jnp.maximum(by1, y1v)
                    xx2 = jnp.minimum(bx2, x2v)
                    yy2 = jnp.minimum(by2, y2v)
                    inter = (jnp.maximum(xx2 - xx1, 0.0)
                             * jnp.maximum(yy2 - yy1, 0.0))
                    areas = (x2v - x1v) * (y2v - y1v)
                    iou = inter / (barea + areas - inter)
                    supp = (iou > NMS_THRESH) | ((i * 16 + iota) == irow)
                    snew = jnp.where(supp, NEG, sv)
                    row0[sl] = snew
                    return jnp.maximum(runmax, snew)

                runmax = lax.fori_loop(0, nv, ibody, neg1)
                newmax = jnp.max(runmax)
                m_new = tuple(
                    jnp.where(((iota + 16 * j) == cstar) & cond, newmax,
                              m_state[j])
                    for j in range(6))

                payload = (jnp.where(iota == 0, bx1, 0.0)
                           + jnp.where(iota == 1, by1, 0.0)
                           + jnp.where(iota == 2, bx2, 0.0)
                           + jnp.where(iota == 3, by2, 0.0)
                           + jnp.where(iota == 4, vstar, 0.0)
                           + jnp.where(iota == 5, cstar.astype(f32), 0.0))
                plsc.store_scatter(out_buf, [t * 8 + iota], payload,
                                   mask=(iota < 8) & cond)

                def wbody(kk, _):
                    pltpu.sync_copy(
                        row0.at[pl.ds(kk * 512, 512)],
                        spm_s.at[pl.ds(pl.multiple_of(off + kk * 512, 512),
                                       512)])
                    return 0

                lax.fori_loop(0, nch, wbody, 0)
                return m_new

            lax.fori_loop(0, DETS_PER_IMG, it, tuple(m_j))
            pltpu.sync_copy(out_buf, out_hbm)


@functools.partial(jax.jit)
def kernel(class_logits, box_regression, proposals):
    f32 = jnp.float32
    lt = jnp.zeros((C_PAD, P_PAD), f32)
    lt = lt.at[:NUM_CLASSES, :N].set(class_logits.T)
    lt = lt.reshape(C_PAD, P_SUB, P_LANE)

    br = box_regression.reshape(N, NUM_CLASSES, 4)
    planes = []
    for k in range(4):
        pk = jnp.zeros((C_PAD, P_PAD), f32)
        pk = pk.at[:NUM_CLASSES, :N].set(br[:, :, k].T)
        planes.append(pk.reshape(C_PAD, P_SUB, P_LANE))
    dx_t, dy_t, dw_t, dh_t = planes

    prop = jnp.zeros((4, P_PAD), f32)
    prop = prop.at[:, :N].set(proposals.T)
    prop = prop.reshape(4, P_SUB, P_LANE)

    plane = jax.ShapeDtypeStruct((C_PAD, P_SUB, P_LANE), f32)
    vec = jax.ShapeDtypeStruct((8, 128), f32)
    s, x1, y1, x2, y2, cnts, maxes = pl.pallas_call(
        _decode_body,
        out_shape=[plane] * 5 + [vec, vec],
    )(lt, dx_t, dy_t, dw_t, dh_t, prop)
    cnts = cnts.reshape(1024)
    maxes = maxes.reshape(1024)

    s2 = s.reshape(C_PAD, P_PAD)
    x12 = x1.reshape(C_PAD, P_PAD)
    y12 = y1.reshape(C_PAD, P_PAD)
    x22 = x2.reshape(C_PAD, P_PAD)
    y22 = y2.reshape(C_PAD, P_PAD)

    mesh = plsc.VectorSubcoreMesh(core_axis_name="c", subcore_axis_name="s")
    sc_nms = pl.kernel(
        _sc_nms_body,
        out_type=jax.ShapeDtypeStruct((832,), f32),
        compiler_params=pltpu.CompilerParams(needs_layout_passes=False),
        mesh=mesh,
        scratch_types=[
            pltpu.VMEM((P_PAD,), f32),
            pltpu.VMEM((P_PAD,), f32),
            pltpu.VMEM((P_PAD,), f32),
            pltpu.VMEM((P_PAD,), f32),
            pltpu.VMEM((P_PAD,), f32),
            pltpu.VMEM((P_PAD,), f32),
            pltpu.VMEM((P_PAD,), f32),
            pltpu.VMEM((P_PAD,), f32),
            pltpu.VMEM((P_PAD,), f32),
            pltpu.VMEM((P_PAD,), f32),
            pltpu.VMEM((832,), f32),
            pltpu.VMEM((128,), f32),
            pltpu.VMEM((128,), f32),
            pltpu.VMEM_SHARED((CAP,), f32),
            pltpu.VMEM_SHARED((CAP,), f32),
            pltpu.VMEM_SHARED((CAP,), f32),
            pltpu.VMEM_SHARED((CAP,), f32),
            pltpu.VMEM_SHARED((CAP,), f32),
            pltpu.SemaphoreType.DMA,
        ],
    )
    packed = sc_nms(s2, x12, y12, x22, y22, cnts, maxes).reshape(104, 8)

    out_boxes = packed[:DETS_PER_IMG, 0:4]
    out_scores = packed[:DETS_PER_IMG, 4]
    out_labels = packed[:DETS_PER_IMG, 5].astype(jnp.int32)
    return out_boxes, out_scores, out_labels


# speedup vs baseline: 1.1432x; 1.1432x over previous
import functools
import math

import jax
import jax.numpy as jnp
from jax import lax
from jax.experimental import pallas as pl
from jax.experimental.pallas import tpu as pltpu
from jax.experimental.pallas import tpu_sc as plsc

N = 5000
NUM_CLASSES = 91
C_PAD = 96
P_SUB = 8
P_LANE = 640
P_PAD = P_SUB * P_LANE
SCORE_THRESH = 0.05
NMS_THRESH = 0.5
DETS_PER_IMG = 100
IMG_H = 800.0
IMG_W = 800.0
BBOX_CLIP = math.log(1000.0 / 16.0)
NEG = -1.0
CAP = 163840
BIG = 2 ** 30


def _decode_body(logits_ref, dx_ref, dy_ref, dw_ref, dh_ref, prop_ref,
                 s_ref, x1_ref, y1_ref, x2_ref, y2_ref, cnt_ref, max_ref):
    px1 = prop_ref[0]
    py1 = prop_ref[1]
    px2 = prop_ref[2]
    py2 = prop_ref[3]
    w = px2 - px1
    h = py2 - py1
    cx = px1 + 0.5 * w
    cy = py1 + 0.5 * h

    logits = logits_ref[...]
    lmax = jnp.max(logits[:NUM_CLASSES], axis=0)
    e = jnp.exp(logits - lmax[None])
    denom = jnp.sum(e[:NUM_CLASSES], axis=0)
    scores = e / denom[None]

    dx = dx_ref[...] * (1.0 / 10.0)
    dy = dy_ref[...] * (1.0 / 10.0)
    dw = jnp.minimum(dw_ref[...] * (1.0 / 5.0), BBOX_CLIP)
    dh = jnp.minimum(dh_ref[...] * (1.0 / 5.0), BBOX_CLIP)
    pcx = dx * w[None] + cx[None]
    pcy = dy * h[None] + cy[None]
    pw = jnp.exp(dw) * w[None]
    ph = jnp.exp(dh) * h[None]
    x1 = jnp.clip(pcx - 0.5 * pw, 0.0, IMG_W)
    y1 = jnp.clip(pcy - 0.5 * ph, 0.0, IMG_H)
    x2 = jnp.clip(pcx + 0.5 * pw, 0.0, IMG_W)
    y2 = jnp.clip(pcy + 0.5 * ph, 0.0, IMG_H)

    pidx = (lax.broadcasted_iota(jnp.int32, (P_SUB, P_LANE), 0) * P_LANE
            + lax.broadcasted_iota(jnp.int32, (P_SUB, P_LANE), 1))
    crow = lax.broadcasted_iota(jnp.int32, (C_PAD, P_SUB, P_LANE), 0)
    valid = ((scores > SCORE_THRESH)
             & ((x2 - x1) >= 0.01)
             & ((y2 - y1) >= 0.01)
             & (pidx[None] < N)
             & (crow >= 1) & (crow < NUM_CLASSES))
    s = jnp.where(valid, scores, NEG)
    s_ref[...] = s
    x1_ref[...] = x1
    y1_ref[...] = y1
    x2_ref[...] = x2
    y2_ref[...] = y2

    cnts = jnp.sum(jnp.sum(jnp.where(valid, 1.0, 0.0), axis=2), axis=1)
    rowmax = jnp.max(jnp.max(s, axis=2), axis=1)
    cnt_row = jnp.concatenate(
        [cnts, jnp.zeros((128 - C_PAD,), jnp.float32)], axis=0).reshape(1, 128)
    max_row = jnp.concatenate(
        [rowmax, jnp.full((128 - C_PAD,), NEG, jnp.float32)],
        axis=0).reshape(1, 128)
    cnt_ref[...] = jnp.concatenate(
        [cnt_row, jnp.zeros((7, 128), jnp.float32)], axis=0)
    max_ref[...] = jnp.concatenate(
        [max_row, jnp.full((7, 128), NEG, jnp.float32)], axis=0)


def _sc_nms_body(s_hbm, x1_hbm, y1_hbm, x2_hbm, y2_hbm, cnt_hbm, max_hbm,
                 out_hbm,
                 row0, row1, row2, row3, row4,
                 cs0, cs1, cs2, cs3, cs4,
                 out_buf, cntf_loc, maxf_loc,
                 spm_s, spm_x1, spm_y1, spm_x2, spm_y2,
                 sem):
    f32 = jnp.float32
    i32 = jnp.int32
    cid = lax.axis_index("c")
    sid = lax.axis_index("s")
    iota = lax.iota(i32, 16)
    neg1 = jnp.full((16,), NEG, f32)
    zero16f = jnp.zeros((16,), f32)
    spms = (spm_s, spm_x1, spm_y1, spm_x2, spm_y2)
    hbms = (s_hbm, x1_hbm, y1_hbm, x2_hbm, y2_hbm)
    rows = (row0, row1, row2, row3, row4)
    css = (cs0, cs1, cs2, cs3, cs4)
    NVR = P_PAD // 16

    @pl.when(cid == 0)
    def _core0():
        pltpu.sync_copy(cnt_hbm.at[pl.ds(0, 128)], cntf_loc)
        pltpu.sync_copy(max_hbm.at[pl.ds(0, 128)], maxf_loc)
        ncnt_j = []
        cnt16_j = []
        off_j = []
        carry = jnp.int32(0)
        for j in range(6):
            n = cntf_loc[pl.ds(j * 16, 16)].astype(i32)
            n = jnp.minimum(jnp.maximum(n, 0), N)
            n16 = (n + 15) & -16
            capj = (n16 + 511) & -512
            cums = plsc.cumsum(capj)
            off_j.append(carry + cums - capj)
            carry = carry + jnp.max(cums)
            ncnt_j.append(n)
            cnt16_j.append(n16)

        for k in range(6):
            n_c = jnp.max(jnp.where(iota == sid, ncnt_j[k], 0))
            n16c = jnp.max(jnp.where(iota == sid, cnt16_j[k], 0))
            off_c = pl.multiple_of(jnp.minimum(
                jnp.max(jnp.where(iota == sid, off_j[k], 0)),
                CAP - P_PAD), 512)

            @pl.when(n16c > 0)
            def _compact(k=k, n_c=n_c, n16c=n16c, off_c=off_c):
                c = k * 16 + sid
                hs = [pltpu.async_copy(hbms[p].at[c], rows[p], sem)
                      for p in range(5)]
                for hnd in hs:
                    hnd.wait()

                def kbody(i, nloc):
                    sv = row0[pl.ds(i * 16, 16)]
                    mask = sv > 0.0
                    ps = plsc.cumsum(jnp.where(mask, 1, 0))
                    pos = nloc + ps - 1
                    for p in range(5):
                        v = rows[p][pl.ds(i * 16, 16)]
                        plsc.store_scatter(css[p], [pos], v, mask=mask)
                    return nloc + plsc.all_reduce_population_count(mask)

                lax.fori_loop(0, NVR, kbody, jnp.zeros((16,), i32))
                padidx = n_c + iota
                plsc.store_scatter(cs0, [padidx], neg1, mask=padidx < n16c)
                nch = (n16c + 511) >> 9

                def dbody(kk, _):
                    for p in range(5):
                        pltpu.sync_copy(
                            css[p].at[pl.ds(kk * 512, 512)],
                            spms[p].at[pl.ds(
                                pl.multiple_of(off_c + kk * 512, 512), 512)])
                    return 0

                lax.fori_loop(0, nch, dbody, 0)

        plsc.subcore_barrier()

        @pl.when(sid == 0)
        def _tile0():
            def zbody(i, _):
                out_buf[pl.ds(i * 16, 16)] = zero16f
                return 0
            lax.fori_loop(0, 52, zbody, 0)

            m_j = [maxf_loc[pl.ds(j * 16, 16)] for j in range(6)]
            enc_j = [off_j[j] * 8192 + cnt16_j[j] for j in range(6)]

            def it(t, m_state):
                vm = m_state[0]
                for j in range(1, 6):
                    vm = jnp.maximum(vm, m_state[j])
                vstar = jnp.max(vm)
                cond = vstar > 0.0
                csel = jnp.full((16,), BIG, i32)
                encv = jnp.zeros((16,), i32)
                for j in range(6):
                    hit = m_state[j] == vstar
                    csel = jnp.minimum(csel,
                                       jnp.where(hit, iota + 16 * j, BIG))
                cstar = jnp.min(csel)
                for j in range(6):
                    at_c = (iota + 16 * j) == cstar
                    encv = encv + jnp.where(at_c, enc_j[j], 0)
                enc = jnp.max(encv)
                n16 = jnp.minimum(enc & 8191, P_PAD)
                off = pl.multiple_of(
                    jnp.minimum(enc >> 13, CAP - P_PAD), 512)
                nch = (n16 + 511) >> 9
                nv = n16 >> 4

                def fetch(kk, _):
                    hs = [pltpu.async_copy(
                        spms[p].at[pl.ds(pl.multiple_of(off + kk * 512, 512),
                                         512)],
                        rows[p].at[pl.ds(kk * 512, 512)], sem)
                        for p in range(5)]
                    for hnd in hs:
                        hnd.wait()
                    return 0

                lax.fori_loop(0, nch, fetch, 0)

                def abody(i, best):
                    sv = row0[pl.ds(i * 16, 16)]
                    return jnp.minimum(
                        best, jnp.where(sv == vstar, i * 16 + iota, BIG))

                bestv = lax.fori_loop(0, nv, abody,
                                      jnp.full((16,), BIG, i32))
                irow = jnp.minimum(jnp.min(bestv), P_PAD - 16)

                irow_splat = jnp.full((16,), irow, i32)
                bx1 = plsc.load_gather(row1, [irow_splat])
                by1 = plsc.load_gather(row2, [irow_splat])
                bx2 = plsc.load_gather(row3, [irow_splat])
                by2 = plsc.load_gather(row4, [irow_splat])
                barea = (bx2 - bx1) * (by2 - by1)

                def ibody(i, runmax):
                    sl = pl.ds(i * 16, 16)
                    sv = row0[sl]
                    x1v = row1[sl]
                    y1v = row2[sl]
                    x2v = row3[sl]
                    y2v = row4[sl]
                    xx1 = jnp.maximum(bx1, x1v)
                    yy1 = jnp.maximum(by1, y1v)
                    xx2 = jnp.minimum(bx2, x2v)
                    yy2 = jnp.minimum(by2, y2v)
                    inter = (jnp.maximum(xx2 - xx1, 0.0)
                             * jnp.maximum(yy2 - yy1, 0.0))
                    areas = (x2v - x1v) * (y2v - y1v)
                    iou = inter / (barea + areas - inter)
                    supp = (iou > NMS_THRESH) | ((i * 16 + iota) == irow)
                    snew = jnp.where(supp, NEG, sv)
                    row0[sl] = snew
                    return jnp.maximum(runmax, snew)

                runmax = lax.fori_loop(0, nv, ibody, neg1)
                newmax = jnp.max(runmax)
                m_new = tuple(
                    jnp.where(((iota + 16 * j) == cstar) & cond, newmax,
                              m_state[j])
                    for j in range(6))

                payload = (jnp.where(iota == 0, bx1, 0.0)
                           + jnp.where(iota == 1, by1, 0.0)
                           + jnp.where(iota == 2, bx2, 0.0)
                           + jnp.where(iota == 3, by2, 0.0)
                           + jnp.where(iota == 4, vstar, 0.0)
                           + jnp.where(iota == 5, cstar.astype(f32), 0.0))
                plsc.store_scatter(out_buf, [t * 8 + iota], payload,
                                   mask=(iota < 8) & cond)

                def wbody(kk, _):
                    pltpu.sync_copy(
                        row0.at[pl.ds(kk * 512, 512)],
                        spm_s.at[pl.ds(pl.multiple_of(off + kk * 512, 512),
                                       512)])
                    return 0

                lax.fori_loop(0, nch, wbody, 0)
                return m_new

            lax.fori_loop(0, DETS_PER_IMG, it, tuple(m_j))
            pltpu.sync_copy(out_buf, out_hbm)


@functools.partial(jax.jit)
def kernel(class_logits, box_regression, proposals):
    f32 = jnp.float32
    lt = jnp.zeros((C_PAD, P_PAD), f32)
    lt = lt.at[:NUM_CLASSES, :N].set(class_logits.T)
    lt = lt.reshape(C_PAD, P_SUB, P_LANE)

    br = box_regression.reshape(N, NUM_CLASSES, 4)
    planes = []
    for k in range(4):
        pk = jnp.zeros((C_PAD, P_PAD), f32)
        pk = pk.at[:NUM_CLASSES, :N].set(br[:, :, k].T)
        planes.append(pk.reshape(C_PAD, P_SUB, P_LANE))
    dx_t, dy_t, dw_t, dh_t = planes

    prop = jnp.zeros((4, P_PAD), f32)
    prop = prop.at[:, :N].set(proposals.T)
    prop = prop.reshape(4, P_SUB, P_LANE)

    plane = jax.ShapeDtypeStruct((C_PAD, P_SUB, P_LANE), f32)
    vec = jax.ShapeDtypeStruct((8, 128), f32)
    s, x1, y1, x2, y2, cnts, maxes = pl.pallas_call(
        _decode_body,
        out_shape=[plane] * 5 + [vec, vec],
    )(lt, dx_t, dy_t, dw_t, dh_t, prop)
    cnts = cnts.reshape(1024)
    maxes = maxes.reshape(1024)

    s2 = s.reshape(C_PAD, P_PAD)
    x12 = x1.reshape(C_PAD, P_PAD)
    y12 = y1.reshape(C_PAD, P_PAD)
    x22 = x2.reshape(C_PAD, P_PAD)
    y22 = y2.reshape(C_PAD, P_PAD)

    mesh = plsc.VectorSubcoreMesh(core_axis_name="c", subcore_axis_name="s")
    sc_nms = pl.kernel(
        _sc_nms_body,
        out_type=jax.ShapeDtypeStruct((832,), f32),
        compiler_params=pltpu.CompilerParams(needs_layout_passes=False),
        mesh=mesh,
        scratch_types=[
            pltpu.VMEM((P_PAD,), f32),
            pltpu.VMEM((P_PAD,), f32),
            pltpu.VMEM((P_PAD,), f32),
            pltpu.VMEM((P_PAD,), f32),
            pltpu.VMEM((P_PAD,), f32),
            pltpu.VMEM((P_PAD,), f32),
            pltpu.VMEM((P_PAD,), f32),
            pltpu.VMEM((P_PAD,), f32),
            pltpu.VMEM((P_PAD,), f32),
            pltpu.VMEM((P_PAD,), f32),
            pltpu.VMEM((832,), f32),
            pltpu.VMEM((128,), f32),
            pltpu.VMEM((128,), f32),
            pltpu.VMEM_SHARED((CAP,), f32),
            pltpu.VMEM_SHARED((CAP,), f32),
            pltpu.VMEM_SHARED((CAP,), f32),
            pltpu.VMEM_SHARED((CAP,), f32),
            pltpu.VMEM_SHARED((CAP,), f32),
            pltpu.SemaphoreType.DMA,
        ],
    )
    packed = sc_nms(s2, x12, y12, x22, y22, cnts, maxes).reshape(104, 8)

    out_boxes = packed[:DETS_PER_IMG, 0:4]
    out_scores = packed[:DETS_PER_IMG, 4]
    out_labels = packed[:DETS_PER_IMG, 5].astype(jnp.int32)
    return out_boxes, out_scores, out_labels


# speedup vs baseline: 1.1448x; 1.0013x over previous
import functools
import math

import jax
import jax.numpy as jnp
from jax import lax
from jax.experimental import pallas as pl
from jax.experimental.pallas import tpu as pltpu
from jax.experimental.pallas import tpu_sc as plsc

N = 5000
NUM_CLASSES = 91
C_PAD = 96
P_SUB = 8
P_LANE = 640
P_PAD = P_SUB * P_LANE
SCORE_THRESH = 0.05
NMS_THRESH = 0.5
DETS_PER_IMG = 100
IMG_H = 800.0
IMG_W = 800.0
BBOX_CLIP = math.log(1000.0 / 16.0)
NEG = -1.0
CAP = 163840
BIG = 2 ** 30


def _decode_body(logits_ref, dx_ref, dy_ref, dw_ref, dh_ref, prop_ref,
                 s_ref, x1_ref, y1_ref, x2_ref, y2_ref, cnt_ref, max_ref):
    px1 = prop_ref[0]
    py1 = prop_ref[1]
    px2 = prop_ref[2]
    py2 = prop_ref[3]
    w = px2 - px1
    h = py2 - py1
    cx = px1 + 0.5 * w
    cy = py1 + 0.5 * h

    logits = logits_ref[...]
    lmax = jnp.max(logits[:NUM_CLASSES], axis=0)
    e = jnp.exp(logits - lmax[None])
    denom = jnp.sum(e[:NUM_CLASSES], axis=0)
    scores = e / denom[None]

    dx = dx_ref[...] * (1.0 / 10.0)
    dy = dy_ref[...] * (1.0 / 10.0)
    dw = jnp.minimum(dw_ref[...] * (1.0 / 5.0), BBOX_CLIP)
    dh = jnp.minimum(dh_ref[...] * (1.0 / 5.0), BBOX_CLIP)
    pcx = dx * w[None] + cx[None]
    pcy = dy * h[None] + cy[None]
    pw = jnp.exp(dw) * w[None]
    ph = jnp.exp(dh) * h[None]
    x1 = jnp.clip(pcx - 0.5 * pw, 0.0, IMG_W)
    y1 = jnp.clip(pcy - 0.5 * ph, 0.0, IMG_H)
    x2 = jnp.clip(pcx + 0.5 * pw, 0.0, IMG_W)
    y2 = jnp.clip(pcy + 0.5 * ph, 0.0, IMG_H)

    pidx = (lax.broadcasted_iota(jnp.int32, (P_SUB, P_LANE), 0) * P_LANE
            + lax.broadcasted_iota(jnp.int32, (P_SUB, P_LANE), 1))
    crow = lax.broadcasted_iota(jnp.int32, (C_PAD, P_SUB, P_LANE), 0)
    valid = ((scores > SCORE_THRESH)
             & ((x2 - x1) >= 0.01)
             & ((y2 - y1) >= 0.01)
             & (pidx[None] < N)
             & (crow >= 1) & (crow < NUM_CLASSES))
    s = jnp.where(valid, scores, NEG)
    s_ref[...] = s
    x1_ref[...] = x1
    y1_ref[...] = y1
    x2_ref[...] = x2
    y2_ref[...] = y2

    cnts = jnp.sum(jnp.sum(jnp.where(valid, 1.0, 0.0), axis=2), axis=1)
    rowmax = jnp.max(jnp.max(s, axis=2), axis=1)
    cnt_row = jnp.concatenate(
        [cnts, jnp.zeros((128 - C_PAD,), jnp.float32)], axis=0).reshape(1, 128)
    max_row = jnp.concatenate(
        [rowmax, jnp.full((128 - C_PAD,), NEG, jnp.float32)],
        axis=0).reshape(1, 128)
    cnt_ref[...] = jnp.concatenate(
        [cnt_row, jnp.zeros((7, 128), jnp.float32)], axis=0)
    max_ref[...] = jnp.concatenate(
        [max_row, jnp.full((7, 128), NEG, jnp.float32)], axis=0)


def _sc_nms_body(s_hbm, x1_hbm, y1_hbm, x2_hbm, y2_hbm, cnt_hbm, max_hbm,
                 out_hbm,
                 row0, row1, row2, row3, row4,
                 cs0, cs1, cs2, cs3, cs4,
                 out_buf, cntf_loc, maxf_loc,
                 spm_s, spm_x1, spm_y1, spm_x2, spm_y2,
                 sem):
    f32 = jnp.float32
    i32 = jnp.int32
    cid = lax.axis_index("c")
    sid = lax.axis_index("s")
    iota = lax.iota(i32, 16)
    neg1 = jnp.full((16,), NEG, f32)
    zero16f = jnp.zeros((16,), f32)
    spms = (spm_s, spm_x1, spm_y1, spm_x2, spm_y2)
    hbms = (s_hbm, x1_hbm, y1_hbm, x2_hbm, y2_hbm)
    rows = (row0, row1, row2, row3, row4)
    css = (cs0, cs1, cs2, cs3, cs4)
    NVR = P_PAD // 16

    @pl.when(cid == 0)
    def _core0():
        pltpu.sync_copy(cnt_hbm.at[pl.ds(0, 128)], cntf_loc)
        pltpu.sync_copy(max_hbm.at[pl.ds(0, 128)], maxf_loc)
        ncnt_j = []
        cnt16_j = []
        off_j = []
        carry = jnp.int32(0)
        for j in range(6):
            n = cntf_loc[pl.ds(j * 16, 16)].astype(i32)
            n = jnp.minimum(jnp.maximum(n, 0), N)
            n16 = (n + 15) & -16
            capj = (n16 + 511) & -512
            cums = plsc.cumsum(capj)
            off_j.append(carry + cums - capj)
            carry = carry + jnp.max(cums)
            ncnt_j.append(n)
            cnt16_j.append(n16)

        for k in range(6):
            n_c = jnp.max(jnp.where(iota == sid, ncnt_j[k], 0))
            n16c = jnp.max(jnp.where(iota == sid, cnt16_j[k], 0))
            off_c = pl.multiple_of(jnp.minimum(
                jnp.max(jnp.where(iota == sid, off_j[k], 0)),
                CAP - P_PAD), 512)

            @pl.when(n16c > 0)
            def _compact(k=k, n_c=n_c, n16c=n16c, off_c=off_c):
                c = k * 16 + sid
                hs = [pltpu.async_copy(hbms[p].at[c], rows[p], sem)
                      for p in range(5)]
                for hnd in hs:
                    hnd.wait()

                def kbody(i, nloc):
                    for u in range(4):
                        sv = row0[pl.ds(i * 64 + u * 16, 16)]
                        mask = sv > 0.0
                        ps = plsc.cumsum(jnp.where(mask, 1, 0))
                        pos = nloc + ps - 1
                        for p in range(5):
                            v = rows[p][pl.ds(i * 64 + u * 16, 16)]
                            plsc.store_scatter(css[p], [pos], v, mask=mask)
                        nloc = nloc + plsc.all_reduce_population_count(mask)
                    return nloc

                lax.fori_loop(0, NVR // 4, kbody, jnp.zeros((16,), i32))
                padidx = n_c + iota
                plsc.store_scatter(cs0, [padidx], neg1, mask=padidx < n16c)
                nch = (n16c + 511) >> 9

                def dbody(kk, _):
                    for p in range(5):
                        pltpu.sync_copy(
                            css[p].at[pl.ds(kk * 512, 512)],
                            spms[p].at[pl.ds(
                                pl.multiple_of(off_c + kk * 512, 512), 512)])
                    return 0

                lax.fori_loop(0, nch, dbody, 0)

        plsc.subcore_barrier()

        @pl.when(sid == 0)
        def _tile0():
            def zbody(i, _):
                out_buf[pl.ds(i * 16, 16)] = zero16f
                return 0
            lax.fori_loop(0, 52, zbody, 0)

            m_j = [maxf_loc[pl.ds(j * 16, 16)] for j in range(6)]
            enc_j = [off_j[j] * 8192 + cnt16_j[j] for j in range(6)]

            def it(t, m_state):
                vm = m_state[0]
                for j in range(1, 6):
                    vm = jnp.maximum(vm, m_state[j])
                vstar = jnp.max(vm)
                cond = vstar > 0.0
                csel = jnp.full((16,), BIG, i32)
                encv = jnp.zeros((16,), i32)
                for j in range(6):
                    hit = m_state[j] == vstar
                    csel = jnp.minimum(csel,
                                       jnp.where(hit, iota + 16 * j, BIG))
                cstar = jnp.min(csel)
                for j in range(6):
                    at_c = (iota + 16 * j) == cstar
                    encv = encv + jnp.where(at_c, enc_j[j], 0)
                enc = jnp.max(encv)
                n16 = jnp.minimum(enc & 8191, P_PAD)
                off = pl.multiple_of(
                    jnp.minimum(enc >> 13, CAP - P_PAD), 512)
                nch = (n16 + 511) >> 9
                nv = n16 >> 4

                def fetch(kk, _):
                    hs = [pltpu.async_copy(
                        spms[p].at[pl.ds(pl.multiple_of(off + kk * 512, 512),
                                         512)],
                        rows[p].at[pl.ds(kk * 512, 512)], sem)
                        for p in range(5)]
                    for hnd in hs:
                        hnd.wait()
                    return 0

                lax.fori_loop(0, nch, fetch, 0)

                def abody(i, best):
                    sv = row0[pl.ds(i * 16, 16)]
                    return jnp.minimum(
                        best, jnp.where(sv == vstar, i * 16 + iota, BIG))

                bestv = lax.fori_loop(0, nv, abody,
                                      jnp.full((16,), BIG, i32))
                irow = jnp.minimum(jnp.min(bestv), P_PAD - 16)

                irow_splat = jnp.full((16,), irow, i32)
                bx1 = plsc.load_gather(row1, [irow_splat])
                by1 = plsc.load_gather(row2, [irow_splat])
                bx2 = plsc.load_gather(row3, [irow_splat])
                by2 = plsc.load_gather(row4, [irow_splat])
                barea = (bx2 - bx1) * (by2 - by1)

                def ibody(i, runmax):
                    sl = pl.ds(i * 16, 16)
                    sv = row0[sl]
                    x1v = row1[sl]
                    y1v = row2[sl]
                    x2v = row3[sl]
                    y2v = row4[sl]
                    xx1 = jnp.maximum(bx1, x1v)
                    yy1 = jnp.maximum(by1, y1v)
                    xx2 = jnp.minimum(bx2, x2v)
                    yy2 = jnp.minimum(by2, y2v)
                    inter = (jnp.maximum(xx2 - xx1, 0.0)
                             * jnp.maximum(yy2 - yy1, 0.0))
                    areas = (x2v - x1v) * (y2v - y1v)
                    iou = inter / (barea + areas - inter)
                    supp = (iou > NMS_THRESH) | ((i * 16 + iota) == irow)
                    snew = jnp.where(supp, NEG, sv)
                    row0[sl] = snew
                    return jnp.maximum(runmax, snew)

                runmax = lax.fori_loop(0, nv, ibody, neg1)
                newmax = jnp.max(runmax)
                m_new = tuple(
                    jnp.where(((iota + 16 * j) == cstar) & cond, newmax,
                              m_state[j])
                    for j in range(6))

                payload = (jnp.where(iota == 0, bx1, 0.0)
                           + jnp.where(iota == 1, by1, 0.0)
                           + jnp.where(iota == 2, bx2, 0.0)
                           + jnp.where(iota == 3, by2, 0.0)
                           + jnp.where(iota == 4, vstar, 0.0)
                           + jnp.where(iota == 5, cstar.astype(f32), 0.0))
                plsc.store_scatter(out_buf, [t * 8 + iota], payload,
                                   mask=(iota < 8) & cond)

                def wbody(kk, _):
                    pltpu.sync_copy(
                        row0.at[pl.ds(kk * 512, 512)],
                        spm_s.at[pl.ds(pl.multiple_of(off + kk * 512, 512),
                                       512)])
                    return 0

                lax.fori_loop(0, nch, wbody, 0)
                return m_new

            lax.fori_loop(0, DETS_PER_IMG, it, tuple(m_j))
            pltpu.sync_copy(out_buf, out_hbm)


@functools.partial(jax.jit)
def kernel(class_logits, box_regression, proposals):
    f32 = jnp.float32
    lt = jnp.zeros((C_PAD, P_PAD), f32)
    lt = lt.at[:NUM_CLASSES, :N].set(class_logits.T)
    lt = lt.reshape(C_PAD, P_SUB, P_LANE)

    br = box_regression.reshape(N, NUM_CLASSES, 4)
    planes = []
    for k in range(4):
        pk = jnp.zeros((C_PAD, P_PAD), f32)
        pk = pk.at[:NUM_CLASSES, :N].set(br[:, :, k].T)
        planes.append(pk.reshape(C_PAD, P_SUB, P_LANE))
    dx_t, dy_t, dw_t, dh_t = planes

    prop = jnp.zeros((4, P_PAD), f32)
    prop = prop.at[:, :N].set(proposals.T)
    prop = prop.reshape(4, P_SUB, P_LANE)

    plane = jax.ShapeDtypeStruct((C_PAD, P_SUB, P_LANE), f32)
    vec = jax.ShapeDtypeStruct((8, 128), f32)
    s, x1, y1, x2, y2, cnts, maxes = pl.pallas_call(
        _decode_body,
        out_shape=[plane] * 5 + [vec, vec],
    )(lt, dx_t, dy_t, dw_t, dh_t, prop)
    cnts = cnts.reshape(1024)
    maxes = maxes.reshape(1024)

    s2 = s.reshape(C_PAD, P_PAD)
    x12 = x1.reshape(C_PAD, P_PAD)
    y12 = y1.reshape(C_PAD, P_PAD)
    x22 = x2.reshape(C_PAD, P_PAD)
    y22 = y2.reshape(C_PAD, P_PAD)

    mesh = plsc.VectorSubcoreMesh(core_axis_name="c", subcore_axis_name="s")
    sc_nms = pl.kernel(
        _sc_nms_body,
        out_type=jax.ShapeDtypeStruct((832,), f32),
        compiler_params=pltpu.CompilerParams(needs_layout_passes=False),
        mesh=mesh,
        scratch_types=[
            pltpu.VMEM((P_PAD,), f32),
            pltpu.VMEM((P_PAD,), f32),
            pltpu.VMEM((P_PAD,), f32),
            pltpu.VMEM((P_PAD,), f32),
            pltpu.VMEM((P_PAD,), f32),
            pltpu.VMEM((P_PAD,), f32),
            pltpu.VMEM((P_PAD,), f32),
            pltpu.VMEM((P_PAD,), f32),
            pltpu.VMEM((P_PAD,), f32),
            pltpu.VMEM((P_PAD,), f32),
            pltpu.VMEM((832,), f32),
            pltpu.VMEM((128,), f32),
            pltpu.VMEM((128,), f32),
            pltpu.VMEM_SHARED((CAP,), f32),
            pltpu.VMEM_SHARED((CAP,), f32),
            pltpu.VMEM_SHARED((CAP,), f32),
            pltpu.VMEM_SHARED((CAP,), f32),
            pltpu.VMEM_SHARED((CAP,), f32),
            pltpu.SemaphoreType.DMA,
        ],
    )
    packed = sc_nms(s2, x12, y12, x22, y22, cnts, maxes).reshape(104, 8)

    out_boxes = packed[:DETS_PER_IMG, 0:4]
    out_scores = packed[:DETS_PER_IMG, 4]
    out_labels = packed[:DETS_PER_IMG, 5].astype(jnp.int32)
    return out_boxes, out_scores, out_labels


# speedup vs baseline: 1.1898x; 1.0393x over previous
import functools
import math

import jax
import jax.numpy as jnp
from jax import lax
from jax.experimental import pallas as pl
from jax.experimental.pallas import tpu as pltpu
from jax.experimental.pallas import tpu_sc as plsc

N = 5000
NUM_CLASSES = 91
C_PAD = 96
P_SUB = 8
P_LANE = 640
P_PAD = P_SUB * P_LANE
SCORE_THRESH = 0.05
NMS_THRESH = 0.5
DETS_PER_IMG = 100
IMG_H = 800.0
IMG_W = 800.0
BBOX_CLIP = math.log(1000.0 / 16.0)
NEG = -1.0
CAP = 163840
BIG = 2 ** 30


def _decode_body(logits_ref, dx_ref, dy_ref, dw_ref, dh_ref, prop_ref,
                 s_ref, x1_ref, y1_ref, x2_ref, y2_ref, cnt_ref, max_ref):
    px1 = prop_ref[0]
    py1 = prop_ref[1]
    px2 = prop_ref[2]
    py2 = prop_ref[3]
    w = px2 - px1
    h = py2 - py1
    cx = px1 + 0.5 * w
    cy = py1 + 0.5 * h

    logits = logits_ref[...]
    lmax = jnp.max(logits[:NUM_CLASSES], axis=0)
    e = jnp.exp(logits - lmax[None])
    denom = jnp.sum(e[:NUM_CLASSES], axis=0)
    scores = e / denom[None]

    dx = dx_ref[...] * (1.0 / 10.0)
    dy = dy_ref[...] * (1.0 / 10.0)
    dw = jnp.minimum(dw_ref[...] * (1.0 / 5.0), BBOX_CLIP)
    dh = jnp.minimum(dh_ref[...] * (1.0 / 5.0), BBOX_CLIP)
    pcx = dx * w[None] + cx[None]
    pcy = dy * h[None] + cy[None]
    pw = jnp.exp(dw) * w[None]
    ph = jnp.exp(dh) * h[None]
    x1 = jnp.clip(pcx - 0.5 * pw, 0.0, IMG_W)
    y1 = jnp.clip(pcy - 0.5 * ph, 0.0, IMG_H)
    x2 = jnp.clip(pcx + 0.5 * pw, 0.0, IMG_W)
    y2 = jnp.clip(pcy + 0.5 * ph, 0.0, IMG_H)

    pidx = (lax.broadcasted_iota(jnp.int32, (P_SUB, P_LANE), 0) * P_LANE
            + lax.broadcasted_iota(jnp.int32, (P_SUB, P_LANE), 1))
    crow = lax.broadcasted_iota(jnp.int32, (C_PAD, P_SUB, P_LANE), 0)
    valid = ((scores > SCORE_THRESH)
             & ((x2 - x1) >= 0.01)
             & ((y2 - y1) >= 0.01)
             & (pidx[None] < N)
             & (crow >= 1) & (crow < NUM_CLASSES))
    s = jnp.where(valid, scores, NEG)
    s_ref[...] = s
    x1_ref[...] = x1
    y1_ref[...] = y1
    x2_ref[...] = x2
    y2_ref[...] = y2

    cnts = jnp.sum(jnp.sum(jnp.where(valid, 1.0, 0.0), axis=2), axis=1)
    rowmax = jnp.max(jnp.max(s, axis=2), axis=1)
    cnt_row = jnp.concatenate(
        [cnts, jnp.zeros((128 - C_PAD,), jnp.float32)], axis=0).reshape(1, 128)
    max_row = jnp.concatenate(
        [rowmax, jnp.full((128 - C_PAD,), NEG, jnp.float32)],
        axis=0).reshape(1, 128)
    cnt_ref[...] = jnp.concatenate(
        [cnt_row, jnp.zeros((7, 128), jnp.float32)], axis=0)
    max_ref[...] = jnp.concatenate(
        [max_row, jnp.full((7, 128), NEG, jnp.float32)], axis=0)


def _sc_nms_body(s_hbm, x1_hbm, y1_hbm, x2_hbm, y2_hbm, cnt_hbm, max_hbm,
                 out_hbm,
                 row0, row1, row2, row3, row4,
                 rb0, rb1, rb2, rb3, rb4,
                 cs0, cs1, cs2, cs3, cs4,
                 out_buf, cntf_loc, maxf_loc,
                 spm_s, spm_x1, spm_y1, spm_x2, spm_y2,
                 sem):
    f32 = jnp.float32
    i32 = jnp.int32
    cid = lax.axis_index("c")
    sid = lax.axis_index("s")
    iota = lax.iota(i32, 16)
    neg1 = jnp.full((16,), NEG, f32)
    zero16f = jnp.zeros((16,), f32)
    spms = (spm_s, spm_x1, spm_y1, spm_x2, spm_y2)
    hbms = (s_hbm, x1_hbm, y1_hbm, x2_hbm, y2_hbm)
    rows = (row0, row1, row2, row3, row4)
    rowsb = (rb0, rb1, rb2, rb3, rb4)
    css = (cs0, cs1, cs2, cs3, cs4)
    NVR = P_PAD // 16

    @pl.when(cid == 0)
    def _core0():
        pltpu.sync_copy(cnt_hbm.at[pl.ds(0, 128)], cntf_loc)
        pltpu.sync_copy(max_hbm.at[pl.ds(0, 128)], maxf_loc)
        ncnt_j = []
        cnt16_j = []
        off_j = []
        carry = jnp.int32(0)
        for j in range(6):
            n = cntf_loc[pl.ds(j * 16, 16)].astype(i32)
            n = jnp.minimum(jnp.maximum(n, 0), N)
            n16 = (n + 15) & -16
            capj = (n16 + 511) & -512
            cums = plsc.cumsum(capj)
            off_j.append(carry + cums - capj)
            carry = carry + jnp.max(cums)
            ncnt_j.append(n)
            cnt16_j.append(n16)

        fet = [pltpu.async_copy(hbms[p].at[sid], rows[p], sem)
               for p in range(5)]
        for k in range(6):
            n_c = jnp.max(jnp.where(iota == sid, ncnt_j[k], 0))
            n16c = jnp.max(jnp.where(iota == sid, cnt16_j[k], 0))
            off_c = pl.multiple_of(jnp.minimum(
                jnp.max(jnp.where(iota == sid, off_j[k], 0)),
                CAP - P_PAD), 512)
            cur = rows if k % 2 == 0 else rowsb
            for hnd in fet:
                hnd.wait()
            if k < 5:
                nxt = rowsb if k % 2 == 0 else rows
                fet = [pltpu.async_copy(hbms[p].at[(k + 1) * 16 + sid],
                                        nxt[p], sem)
                       for p in range(5)]

            @pl.when(n16c > 0)
            def _compact(k=k, n_c=n_c, n16c=n16c, off_c=off_c, cur=cur):
                def kbody(i, nloc):
                    for u in range(4):
                        sv = cur[0][pl.ds(i * 64 + u * 16, 16)]
                        mask = sv > 0.0
                        ps = plsc.cumsum(jnp.where(mask, 1, 0))
                        pos = nloc + ps - 1
                        for p in range(5):
                            v = cur[p][pl.ds(i * 64 + u * 16, 16)]
                            plsc.store_scatter(css[p], [pos], v, mask=mask)
                        nloc = nloc + plsc.all_reduce_population_count(mask)
                    return nloc

                lax.fori_loop(0, NVR // 4, kbody, jnp.zeros((16,), i32))
                padidx = n_c + iota
                plsc.store_scatter(cs0, [padidx], neg1, mask=padidx < n16c)
                nch = (n16c + 511) >> 9

                def dbody(kk, _):
                    for p in range(5):
                        pltpu.sync_copy(
                            css[p].at[pl.ds(kk * 512, 512)],
                            spms[p].at[pl.ds(
                                pl.multiple_of(off_c + kk * 512, 512), 512)])
                    return 0

                lax.fori_loop(0, nch, dbody, 0)

        plsc.subcore_barrier()

        @pl.when(sid == 0)
        def _tile0():
            def zbody(i, _):
                out_buf[pl.ds(i * 16, 16)] = zero16f
                return 0
            lax.fori_loop(0, 52, zbody, 0)

            m_j = [maxf_loc[pl.ds(j * 16, 16)] for j in range(6)]
            enc_j = [off_j[j] * 8192 + cnt16_j[j] for j in range(6)]

            def it(t, m_state):
                vm = m_state[0]
                for j in range(1, 6):
                    vm = jnp.maximum(vm, m_state[j])
                vstar = jnp.max(vm)
                cond = vstar > 0.0
                csel = jnp.full((16,), BIG, i32)
                encv = jnp.zeros((16,), i32)
                for j in range(6):
                    hit = m_state[j] == vstar
                    csel = jnp.minimum(csel,
                                       jnp.where(hit, iota + 16 * j, BIG))
                cstar = jnp.min(csel)
                for j in range(6):
                    at_c = (iota + 16 * j) == cstar
                    encv = encv + jnp.where(at_c, enc_j[j], 0)
                enc = jnp.max(encv)
                n16 = jnp.minimum(enc & 8191, P_PAD)
                off = pl.multiple_of(
                    jnp.minimum(enc >> 13, CAP - P_PAD), 512)
                nch = (n16 + 511) >> 9
                nv = n16 >> 4

                def fetch(kk, _):
                    hs = [pltpu.async_copy(
                        spms[p].at[pl.ds(pl.multiple_of(off + kk * 512, 512),
                                         512)],
                        rows[p].at[pl.ds(kk * 512, 512)], sem)
                        for p in range(5)]
                    for hnd in hs:
                        hnd.wait()
                    return 0

                lax.fori_loop(0, nch, fetch, 0)

                def abody(i, best):
                    sv = row0[pl.ds(i * 16, 16)]
                    return jnp.minimum(
                        best, jnp.where(sv == vstar, i * 16 + iota, BIG))

                bestv = lax.fori_loop(0, nv, abody,
                                      jnp.full((16,), BIG, i32))
                irow = jnp.minimum(jnp.min(bestv), P_PAD - 16)

                irow_splat = jnp.full((16,), irow, i32)
                bx1 = plsc.load_gather(row1, [irow_splat])
                by1 = plsc.load_gather(row2, [irow_splat])
                bx2 = plsc.load_gather(row3, [irow_splat])
                by2 = plsc.load_gather(row4, [irow_splat])
                barea = (bx2 - bx1) * (by2 - by1)

                def ibody(i, runmax):
                    sl = pl.ds(i * 16, 16)
                    sv = row0[sl]
                    x1v = row1[sl]
                    y1v = row2[sl]
                    x2v = row3[sl]
                    y2v = row4[sl]
                    xx1 = jnp.maximum(bx1, x1v)
                    yy1 = jnp.maximum(by1, y1v)
                    xx2 = jnp.minimum(bx2, x2v)
                    yy2 = jnp.minimum(by2, y2v)
                    inter = (jnp.maximum(xx2 - xx1, 0.0)
                             * jnp.maximum(yy2 - yy1, 0.0))
                    areas = (x2v - x1v) * (y2v - y1v)
                    iou = inter / (barea + areas - inter)
                    supp = (iou > NMS_THRESH) | ((i * 16 + iota) == irow)
                    snew = jnp.where(supp, NEG, sv)
                    row0[sl] = snew
                    return jnp.maximum(runmax, snew)

                runmax = lax.fori_loop(0, nv, ibody, neg1)
                newmax = jnp.max(runmax)
                m_new = tuple(
                    jnp.where(((iota + 16 * j) == cstar) & cond, newmax,
                              m_state[j])
                    for j in range(6))

                payload = (jnp.where(iota == 0, bx1, 0.0)
                           + jnp.where(iota == 1, by1, 0.0)
                           + jnp.where(iota == 2, bx2, 0.0)
                           + jnp.where(iota == 3, by2, 0.0)
                           + jnp.where(iota == 4, vstar, 0.0)
                           + jnp.where(iota == 5, cstar.astype(f32), 0.0))
                plsc.store_scatter(out_buf, [t * 8 + iota], payload,
                                   mask=(iota < 8) & cond)

                def wbody(kk, _):
                    pltpu.sync_copy(
                        row0.at[pl.ds(kk * 512, 512)],
                        spm_s.at[pl.ds(pl.multiple_of(off + kk * 512, 512),
                                       512)])
                    return 0

                lax.fori_loop(0, nch, wbody, 0)
                return m_new

            lax.fori_loop(0, DETS_PER_IMG, it, tuple(m_j))
            pltpu.sync_copy(out_buf, out_hbm)


@functools.partial(jax.jit)
def kernel(class_logits, box_regression, proposals):
    f32 = jnp.float32
    lt = jnp.zeros((C_PAD, P_PAD), f32)
    lt = lt.at[:NUM_CLASSES, :N].set(class_logits.T)
    lt = lt.reshape(C_PAD, P_SUB, P_LANE)

    br = box_regression.reshape(N, NUM_CLASSES, 4)
    planes = []
    for k in range(4):
        pk = jnp.zeros((C_PAD, P_PAD), f32)
        pk = pk.at[:NUM_CLASSES, :N].set(br[:, :, k].T)
        planes.append(pk.reshape(C_PAD, P_SUB, P_LANE))
    dx_t, dy_t, dw_t, dh_t = planes

    prop = jnp.zeros((4, P_PAD), f32)
    prop = prop.at[:, :N].set(proposals.T)
    prop = prop.reshape(4, P_SUB, P_LANE)

    plane = jax.ShapeDtypeStruct((C_PAD, P_SUB, P_LANE), f32)
    vec = jax.ShapeDtypeStruct((8, 128), f32)
    s, x1, y1, x2, y2, cnts, maxes = pl.pallas_call(
        _decode_body,
        out_shape=[plane] * 5 + [vec, vec],
    )(lt, dx_t, dy_t, dw_t, dh_t, prop)
    cnts = cnts.reshape(1024)
    maxes = maxes.reshape(1024)

    s2 = s.reshape(C_PAD, P_PAD)
    x12 = x1.reshape(C_PAD, P_PAD)
    y12 = y1.reshape(C_PAD, P_PAD)
    x22 = x2.reshape(C_PAD, P_PAD)
    y22 = y2.reshape(C_PAD, P_PAD)

    mesh = plsc.VectorSubcoreMesh(core_axis_name="c", subcore_axis_name="s")
    sc_nms = pl.kernel(
        _sc_nms_body,
        out_type=jax.ShapeDtypeStruct((832,), f32),
        compiler_params=pltpu.CompilerParams(needs_layout_passes=False),
        mesh=mesh,
        scratch_types=[
            pltpu.VMEM((P_PAD,), f32),
            pltpu.VMEM((P_PAD,), f32),
            pltpu.VMEM((P_PAD,), f32),
            pltpu.VMEM((P_PAD,), f32),
            pltpu.VMEM((P_PAD,), f32),
            pltpu.VMEM((P_PAD,), f32),
            pltpu.VMEM((P_PAD,), f32),
            pltpu.VMEM((P_PAD,), f32),
            pltpu.VMEM((P_PAD,), f32),
            pltpu.VMEM((P_PAD,), f32),
            pltpu.VMEM((P_PAD,), f32),
            pltpu.VMEM((P_PAD,), f32),
            pltpu.VMEM((P_PAD,), f32),
            pltpu.VMEM((P_PAD,), f32),
            pltpu.VMEM((P_PAD,), f32),
            pltpu.VMEM((832,), f32),
            pltpu.VMEM((128,), f32),
            pltpu.VMEM((128,), f32),
            pltpu.VMEM_SHARED((CAP,), f32),
            pltpu.VMEM_SHARED((CAP,), f32),
            pltpu.VMEM_SHARED((CAP,), f32),
            pltpu.VMEM_SHARED((CAP,), f32),
            pltpu.VMEM_SHARED((CAP,), f32),
            pltpu.SemaphoreType.DMA,
        ],
    )
    packed = sc_nms(s2, x12, y12, x22, y22, cnts, maxes).reshape(104, 8)

    out_boxes = packed[:DETS_PER_IMG, 0:4]
    out_scores = packed[:DETS_PER_IMG, 4]
    out_labels = packed[:DETS_PER_IMG, 5].astype(jnp.int32)
    return out_boxes, out_scores, out_labels


# speedup vs baseline: 1.1988x; 1.0076x over previous
import functools
import math

import jax
import jax.numpy as jnp
from jax import lax
from jax.experimental import pallas as pl
from jax.experimental.pallas import tpu as pltpu
from jax.experimental.pallas import tpu_sc as plsc

N = 5000
NUM_CLASSES = 91
C_PAD = 96
P_SUB = 8
P_LANE = 640
P_PAD = P_SUB * P_LANE
SCORE_THRESH = 0.05
NMS_THRESH = 0.5
DETS_PER_IMG = 100
IMG_H = 800.0
IMG_W = 800.0
BBOX_CLIP = math.log(1000.0 / 16.0)
NEG = -1.0
CAP = 163840
BIG = 2 ** 30


def _decode_body(logits_ref, dx_ref, dy_ref, dw_ref, dh_ref, prop_ref,
                 s_ref, x1_ref, y1_ref, x2_ref, y2_ref, cnt_ref, max_ref):
    px1 = prop_ref[0]
    py1 = prop_ref[1]
    px2 = prop_ref[2]
    py2 = prop_ref[3]
    w = px2 - px1
    h = py2 - py1
    cx = px1 + 0.5 * w
    cy = py1 + 0.5 * h

    logits = logits_ref[...]
    lmax = jnp.max(logits[:NUM_CLASSES], axis=0)
    e = jnp.exp(logits - lmax[None])
    denom = jnp.sum(e[:NUM_CLASSES], axis=0)
    scores = e / denom[None]

    dx = dx_ref[...] * (1.0 / 10.0)
    dy = dy_ref[...] * (1.0 / 10.0)
    dw = jnp.minimum(dw_ref[...] * (1.0 / 5.0), BBOX_CLIP)
    dh = jnp.minimum(dh_ref[...] * (1.0 / 5.0), BBOX_CLIP)
    pcx = dx * w[None] + cx[None]
    pcy = dy * h[None] + cy[None]
    pw = jnp.exp(dw) * w[None]
    ph = jnp.exp(dh) * h[None]
    x1 = jnp.clip(pcx - 0.5 * pw, 0.0, IMG_W)
    y1 = jnp.clip(pcy - 0.5 * ph, 0.0, IMG_H)
    x2 = jnp.clip(pcx + 0.5 * pw, 0.0, IMG_W)
    y2 = jnp.clip(pcy + 0.5 * ph, 0.0, IMG_H)

    pidx = (lax.broadcasted_iota(jnp.int32, (P_SUB, P_LANE), 0) * P_LANE
            + lax.broadcasted_iota(jnp.int32, (P_SUB, P_LANE), 1))
    crow = lax.broadcasted_iota(jnp.int32, (C_PAD, P_SUB, P_LANE), 0)
    valid = ((scores > SCORE_THRESH)
             & ((x2 - x1) >= 0.01)
             & ((y2 - y1) >= 0.01)
             & (pidx[None] < N)
             & (crow >= 1) & (crow < NUM_CLASSES))
    s = jnp.where(valid, scores, NEG)
    s_ref[...] = s
    x1_ref[...] = x1
    y1_ref[...] = y1
    x2_ref[...] = x2
    y2_ref[...] = y2

    cnts = jnp.sum(jnp.sum(jnp.where(valid, 1.0, 0.0), axis=2), axis=1)
    rowmax = jnp.max(jnp.max(s, axis=2), axis=1)
    cnt_row = jnp.concatenate(
        [cnts, jnp.zeros((128 - C_PAD,), jnp.float32)], axis=0).reshape(1, 128)
    max_row = jnp.concatenate(
        [rowmax, jnp.full((128 - C_PAD,), NEG, jnp.float32)],
        axis=0).reshape(1, 128)
    cnt_ref[...] = jnp.concatenate(
        [cnt_row, jnp.zeros((7, 128), jnp.float32)], axis=0)
    max_ref[...] = jnp.concatenate(
        [max_row, jnp.full((7, 128), NEG, jnp.float32)], axis=0)


def _sc_nms_body(s_hbm, x1_hbm, y1_hbm, x2_hbm, y2_hbm, cnt_hbm, max_hbm,
                 out_hbm,
                 row0, row1, row2, row3, row4,
                 rb0, rb1, rb2, rb3, rb4,
                 cs0, cs1, cs2, cs3, cs4,
                 out_buf, cntf_loc, maxf_loc,
                 spm_s, spm_x1, spm_y1, spm_x2, spm_y2,
                 sem):
    f32 = jnp.float32
    i32 = jnp.int32
    cid = lax.axis_index("c")
    sid = lax.axis_index("s")
    iota = lax.iota(i32, 16)
    neg1 = jnp.full((16,), NEG, f32)
    zero16f = jnp.zeros((16,), f32)
    spms = (spm_s, spm_x1, spm_y1, spm_x2, spm_y2)
    hbms = (s_hbm, x1_hbm, y1_hbm, x2_hbm, y2_hbm)
    rows = (row0, row1, row2, row3, row4)
    rowsb = (rb0, rb1, rb2, rb3, rb4)
    css = (cs0, cs1, cs2, cs3, cs4)
    NVR = P_PAD // 16

    @pl.when(cid == 0)
    def _core0():
        pltpu.sync_copy(cnt_hbm.at[pl.ds(0, 128)], cntf_loc)
        pltpu.sync_copy(max_hbm.at[pl.ds(0, 128)], maxf_loc)
        ncnt_j = []
        cnt16_j = []
        off_j = []
        carry = jnp.int32(0)
        for j in range(6):
            n = cntf_loc[pl.ds(j * 16, 16)].astype(i32)
            n = jnp.minimum(jnp.maximum(n, 0), N)
            n16 = (n + 15) & -16
            capj = (n16 + 511) & -512
            cums = plsc.cumsum(capj)
            off_j.append(carry + cums - capj)
            carry = carry + jnp.max(cums)
            ncnt_j.append(n)
            cnt16_j.append(n16)

        fet = [pltpu.async_copy(hbms[p].at[sid], rows[p], sem)
               for p in range(5)]
        for k in range(6):
            n_c = jnp.max(jnp.where(iota == sid, ncnt_j[k], 0))
            n16c = jnp.max(jnp.where(iota == sid, cnt16_j[k], 0))
            off_c = pl.multiple_of(jnp.minimum(
                jnp.max(jnp.where(iota == sid, off_j[k], 0)),
                CAP - P_PAD), 512)
            cur = rows if k % 2 == 0 else rowsb
            for hnd in fet:
                hnd.wait()
            if k < 5:
                nxt = rowsb if k % 2 == 0 else rows
                fet = [pltpu.async_copy(hbms[p].at[(k + 1) * 16 + sid],
                                        nxt[p], sem)
                       for p in range(5)]

            @pl.when(n16c > 0)
            def _compact(k=k, n_c=n_c, n16c=n16c, off_c=off_c, cur=cur):
                def kbody(i, nloc):
                    for u in range(4):
                        sv = cur[0][pl.ds(i * 64 + u * 16, 16)]
                        mask = sv > 0.0
                        ps = plsc.cumsum(jnp.where(mask, 1, 0))
                        pos = nloc + ps - 1
                        for p in range(5):
                            v = cur[p][pl.ds(i * 64 + u * 16, 16)]
                            plsc.store_scatter(css[p], [pos], v, mask=mask)
                        nloc = nloc + plsc.all_reduce_population_count(mask)
                    return nloc

                lax.fori_loop(0, NVR // 4, kbody, jnp.zeros((16,), i32))
                padidx = n_c + iota
                plsc.store_scatter(cs0, [padidx], neg1, mask=padidx < n16c)
                nch = (n16c + 511) >> 9

                def dbody(kk, _):
                    hs = [pltpu.async_copy(
                        css[p].at[pl.ds(kk * 512, 512)],
                        spms[p].at[pl.ds(
                            pl.multiple_of(off_c + kk * 512, 512), 512)],
                        sem) for p in range(5)]
                    for hnd in hs:
                        hnd.wait()
                    return 0

                lax.fori_loop(0, nch, dbody, 0)

        plsc.subcore_barrier()

        @pl.when(sid == 0)
        def _tile0():
            def zbody(i, _):
                out_buf[pl.ds(i * 16, 16)] = zero16f
                return 0
            lax.fori_loop(0, 52, zbody, 0)

            m_j = [maxf_loc[pl.ds(j * 16, 16)] for j in range(6)]
            enc_j = [off_j[j] * 8192 + cnt16_j[j] for j in range(6)]

            def it(t, m_state):
                vm = m_state[0]
                for j in range(1, 6):
                    vm = jnp.maximum(vm, m_state[j])
                vstar = jnp.max(vm)
                cond = vstar > 0.0
                csel = jnp.full((16,), BIG, i32)
                encv = jnp.zeros((16,), i32)
                for j in range(6):
                    hit = m_state[j] == vstar
                    csel = jnp.minimum(csel,
                                       jnp.where(hit, iota + 16 * j, BIG))
                cstar = jnp.min(csel)
                for j in range(6):
                    at_c = (iota + 16 * j) == cstar
                    encv = encv + jnp.where(at_c, enc_j[j], 0)
                enc = jnp.max(encv)
                n16 = jnp.minimum(enc & 8191, P_PAD)
                off = pl.multiple_of(
                    jnp.minimum(enc >> 13, CAP - P_PAD), 512)
                nch = (n16 + 511) >> 9
                nv = n16 >> 4

                def fetch(kk, _):
                    hs = [pltpu.async_copy(
                        spms[p].at[pl.ds(pl.multiple_of(off + kk * 512, 512),
                                         512)],
                        rows[p].at[pl.ds(kk * 512, 512)], sem)
                        for p in range(5)]
                    for hnd in hs:
                        hnd.wait()
                    return 0

                lax.fori_loop(0, nch, fetch, 0)

                def abody(i, best):
                    sv = row0[pl.ds(i * 16, 16)]
                    return jnp.minimum(
                        best, jnp.where(sv == vstar, i * 16 + iota, BIG))

                bestv = lax.fori_loop(0, nv, abody,
                                      jnp.full((16,), BIG, i32))
                irow = jnp.minimum(jnp.min(bestv), P_PAD - 16)

                irow_splat = jnp.full((16,), irow, i32)
                bx1 = plsc.load_gather(row1, [irow_splat])
                by1 = plsc.load_gather(row2, [irow_splat])
                bx2 = plsc.load_gather(row3, [irow_splat])
                by2 = plsc.load_gather(row4, [irow_splat])
                barea = (bx2 - bx1) * (by2 - by1)

                def ibody(i, runmax):
                    sl = pl.ds(i * 16, 16)
                    sv = row0[sl]
                    x1v = row1[sl]
                    y1v = row2[sl]
                    x2v = row3[sl]
                    y2v = row4[sl]
                    xx1 = jnp.maximum(bx1, x1v)
                    yy1 = jnp.maximum(by1, y1v)
                    xx2 = jnp.minimum(bx2, x2v)
                    yy2 = jnp.minimum(by2, y2v)
                    inter = (jnp.maximum(xx2 - xx1, 0.0)
                             * jnp.maximum(yy2 - yy1, 0.0))
                    areas = (x2v - x1v) * (y2v - y1v)
                    iou = inter / (barea + areas - inter)
                    supp = (iou > NMS_THRESH) | ((i * 16 + iota) == irow)
                    snew = jnp.where(supp, NEG, sv)
                    row0[sl] = snew
                    return jnp.maximum(runmax, snew)

                runmax = lax.fori_loop(0, nv, ibody, neg1)
                newmax = jnp.max(runmax)
                m_new = tuple(
                    jnp.where(((iota + 16 * j) == cstar) & cond, newmax,
                              m_state[j])
                    for j in range(6))

                payload = (jnp.where(iota == 0, bx1, 0.0)
                           + jnp.where(iota == 1, by1, 0.0)
                           + jnp.where(iota == 2, bx2, 0.0)
                           + jnp.where(iota == 3, by2, 0.0)
                           + jnp.where(iota == 4, vstar, 0.0)
                           + jnp.where(iota == 5, cstar.astype(f32), 0.0))
                plsc.store_scatter(out_buf, [t * 8 + iota], payload,
                                   mask=(iota < 8) & cond)

                def wbody(kk, _):
                    pltpu.sync_copy(
                        row0.at[pl.ds(kk * 512, 512)],
                        spm_s.at[pl.ds(pl.multiple_of(off + kk * 512, 512),
                                       512)])
                    return 0

                lax.fori_loop(0, nch, wbody, 0)
                return m_new

            lax.fori_loop(0, DETS_PER_IMG, it, tuple(m_j))
            pltpu.sync_copy(out_buf, out_hbm)


@functools.partial(jax.jit)
def kernel(class_logits, box_regression, proposals):
    f32 = jnp.float32
    lt = jnp.zeros((C_PAD, P_PAD), f32)
    lt = lt.at[:NUM_CLASSES, :N].set(class_logits.T)
    lt = lt.reshape(C_PAD, P_SUB, P_LANE)

    br = box_regression.reshape(N, NUM_CLASSES, 4)
    planes = []
    for k in range(4):
        pk = jnp.zeros((C_PAD, P_PAD), f32)
        pk = pk.at[:NUM_CLASSES, :N].set(br[:, :, k].T)
        planes.append(pk.reshape(C_PAD, P_SUB, P_LANE))
    dx_t, dy_t, dw_t, dh_t = planes

    prop = jnp.zeros((4, P_PAD), f32)
    prop = prop.at[:, :N].set(proposals.T)
    prop = prop.reshape(4, P_SUB, P_LANE)

    plane = jax.ShapeDtypeStruct((C_PAD, P_SUB, P_LANE), f32)
    vec = jax.ShapeDtypeStruct((8, 128), f32)
    s, x1, y1, x2, y2, cnts, maxes = pl.pallas_call(
        _decode_body,
        out_shape=[plane] * 5 + [vec, vec],
    )(lt, dx_t, dy_t, dw_t, dh_t, prop)
    cnts = cnts.reshape(1024)
    maxes = maxes.reshape(1024)

    s2 = s.reshape(C_PAD, P_PAD)
    x12 = x1.reshape(C_PAD, P_PAD)
    y12 = y1.reshape(C_PAD, P_PAD)
    x22 = x2.reshape(C_PAD, P_PAD)
    y22 = y2.reshape(C_PAD, P_PAD)

    mesh = plsc.VectorSubcoreMesh(core_axis_name="c", subcore_axis_name="s")
    sc_nms = pl.kernel(
        _sc_nms_body,
        out_type=jax.ShapeDtypeStruct((832,), f32),
        compiler_params=pltpu.CompilerParams(needs_layout_passes=False),
        mesh=mesh,
        scratch_types=[
            pltpu.VMEM((P_PAD,), f32),
            pltpu.VMEM((P_PAD,), f32),
            pltpu.VMEM((P_PAD,), f32),
            pltpu.VMEM((P_PAD,), f32),
            pltpu.VMEM((P_PAD,), f32),
            pltpu.VMEM((P_PAD,), f32),
            pltpu.VMEM((P_PAD,), f32),
            pltpu.VMEM((P_PAD,), f32),
            pltpu.VMEM((P_PAD,), f32),
            pltpu.VMEM((P_PAD,), f32),
            pltpu.VMEM((P_PAD,), f32),
            pltpu.VMEM((P_PAD,), f32),
            pltpu.VMEM((P_PAD,), f32),
            pltpu.VMEM((P_PAD,), f32),
            pltpu.VMEM((P_PAD,), f32),
            pltpu.VMEM((832,), f32),
            pltpu.VMEM((128,), f32),
            pltpu.VMEM((128,), f32),
            pltpu.VMEM_SHARED((CAP,), f32),
            pltpu.VMEM_SHARED((CAP,), f32),
            pltpu.VMEM_SHARED((CAP,), f32),
            pltpu.VMEM_SHARED((CAP,), f32),
            pltpu.VMEM_SHARED((CAP,), f32),
            pltpu.SemaphoreType.DMA,
        ],
    )
    packed = sc_nms(s2, x12, y12, x22, y22, cnts, maxes).reshape(104, 8)

    out_boxes = packed[:DETS_PER_IMG, 0:4]
    out_scores = packed[:DETS_PER_IMG, 4]
    out_labels = packed[:DETS_PER_IMG, 5].astype(jnp.int32)
    return out_boxes, out_scores, out_labels


# speedup vs baseline: 1.2337x; 1.0291x over previous
import functools
import math

import jax
import jax.numpy as jnp
from jax import lax
from jax.experimental import pallas as pl
from jax.experimental.pallas import tpu as pltpu
from jax.experimental.pallas import tpu_sc as plsc

N = 5000
NUM_CLASSES = 91
C_PAD = 96
P_SUB = 8
P_LANE = 640
P_PAD = P_SUB * P_LANE
SCORE_THRESH = 0.05
NMS_THRESH = 0.5
DETS_PER_IMG = 100
IMG_H = 800.0
IMG_W = 800.0
BBOX_CLIP = math.log(1000.0 / 16.0)
NEG = -1.0
CAP = 163840
BIG = 2 ** 30


def _decode_body(logits_ref, dx_ref, dy_ref, dw_ref, dh_ref, prop_ref,
                 s_ref, x1_ref, y1_ref, x2_ref, y2_ref, cnt_ref, max_ref):
    px1 = prop_ref[0]
    py1 = prop_ref[1]
    px2 = prop_ref[2]
    py2 = prop_ref[3]
    w = px2 - px1
    h = py2 - py1
    cx = px1 + 0.5 * w
    cy = py1 + 0.5 * h

    logits = logits_ref[...]
    lmax = jnp.max(logits[:NUM_CLASSES], axis=0)
    e = jnp.exp(logits - lmax[None])
    denom = jnp.sum(e[:NUM_CLASSES], axis=0)
    scores = e / denom[None]

    dx = dx_ref[...] * (1.0 / 10.0)
    dy = dy_ref[...] * (1.0 / 10.0)
    dw = jnp.minimum(dw_ref[...] * (1.0 / 5.0), BBOX_CLIP)
    dh = jnp.minimum(dh_ref[...] * (1.0 / 5.0), BBOX_CLIP)
    pcx = dx * w[None] + cx[None]
    pcy = dy * h[None] + cy[None]
    pw = jnp.exp(dw) * w[None]
    ph = jnp.exp(dh) * h[None]
    x1 = jnp.clip(pcx - 0.5 * pw, 0.0, IMG_W)
    y1 = jnp.clip(pcy - 0.5 * ph, 0.0, IMG_H)
    x2 = jnp.clip(pcx + 0.5 * pw, 0.0, IMG_W)
    y2 = jnp.clip(pcy + 0.5 * ph, 0.0, IMG_H)

    pidx = (lax.broadcasted_iota(jnp.int32, (P_SUB, P_LANE), 0) * P_LANE
            + lax.broadcasted_iota(jnp.int32, (P_SUB, P_LANE), 1))
    crow = lax.broadcasted_iota(jnp.int32, (C_PAD, P_SUB, P_LANE), 0)
    valid = ((scores > SCORE_THRESH)
             & ((x2 - x1) >= 0.01)
             & ((y2 - y1) >= 0.01)
             & (pidx[None] < N)
             & (crow >= 1) & (crow < NUM_CLASSES))
    s = jnp.where(valid, scores, NEG)
    s_ref[...] = s
    x1_ref[...] = x1
    y1_ref[...] = y1
    x2_ref[...] = x2
    y2_ref[...] = y2

    cnts = jnp.sum(jnp.sum(jnp.where(valid, 1.0, 0.0), axis=2), axis=1)
    rowmax = jnp.max(jnp.max(s, axis=2), axis=1)
    cnt_row = jnp.concatenate(
        [cnts, jnp.zeros((128 - C_PAD,), jnp.float32)], axis=0).reshape(1, 128)
    max_row = jnp.concatenate(
        [rowmax, jnp.full((128 - C_PAD,), NEG, jnp.float32)],
        axis=0).reshape(1, 128)
    cnt_ref[...] = jnp.concatenate(
        [cnt_row, jnp.zeros((7, 128), jnp.float32)], axis=0)
    max_ref[...] = jnp.concatenate(
        [max_row, jnp.full((7, 128), NEG, jnp.float32)], axis=0)


def _sc_nms_body(s_hbm, x1_hbm, y1_hbm, x2_hbm, y2_hbm, cnt_hbm, max_hbm,
                 out_hbm,
                 row0, row1, row2, row3, row4,
                 rb0, rb1, rb2, rb3, rb4,
                 cs0, cs1, cs2, cs3, cs4,
                 out_buf, cntf_loc, maxf_loc,
                 spm_s, spm_x1, spm_y1, spm_x2, spm_y2,
                 sem):
    f32 = jnp.float32
    i32 = jnp.int32
    cid = lax.axis_index("c")
    sid = lax.axis_index("s")
    iota = lax.iota(i32, 16)
    neg1 = jnp.full((16,), NEG, f32)
    zero16f = jnp.zeros((16,), f32)
    spms = (spm_s, spm_x1, spm_y1, spm_x2, spm_y2)
    hbms = (s_hbm, x1_hbm, y1_hbm, x2_hbm, y2_hbm)
    rows = (row0, row1, row2, row3, row4)
    rowsb = (rb0, rb1, rb2, rb3, rb4)
    css = (cs0, cs1, cs2, cs3, cs4)
    NVR = P_PAD // 16

    @pl.when(cid == 0)
    def _core0():
        pltpu.sync_copy(cnt_hbm.at[pl.ds(0, 128)], cntf_loc)
        pltpu.sync_copy(max_hbm.at[pl.ds(0, 128)], maxf_loc)
        ncnt_j = []
        cnt16_j = []
        off_j = []
        carry = jnp.int32(0)
        for j in range(6):
            n = cntf_loc[pl.ds(j * 16, 16)].astype(i32)
            n = jnp.minimum(jnp.maximum(n, 0), N)
            n16 = (n + 15) & -16
            capj = (n16 + 511) & -512
            cums = plsc.cumsum(capj)
            off_j.append(carry + cums - capj)
            carry = carry + jnp.max(cums)
            ncnt_j.append(n)
            cnt16_j.append(n16)

        fet = [pltpu.async_copy(hbms[p].at[sid], rows[p], sem)
               for p in range(5)]
        for k in range(6):
            n_c = jnp.max(jnp.where(iota == sid, ncnt_j[k], 0))
            n16c = jnp.max(jnp.where(iota == sid, cnt16_j[k], 0))
            off_c = pl.multiple_of(jnp.minimum(
                jnp.max(jnp.where(iota == sid, off_j[k], 0)),
                CAP - P_PAD), 512)
            cur = rows if k % 2 == 0 else rowsb
            for hnd in fet:
                hnd.wait()
            if k < 5:
                nxt = rowsb if k % 2 == 0 else rows
                fet = [pltpu.async_copy(hbms[p].at[(k + 1) * 16 + sid],
                                        nxt[p], sem)
                       for p in range(5)]

            @pl.when(n16c > 0)
            def _compact(k=k, n_c=n_c, n16c=n16c, off_c=off_c, cur=cur):
                def kbody(i, nloc):
                    for u in range(4):
                        sv = cur[0][pl.ds(i * 64 + u * 16, 16)]
                        mask = sv > 0.0
                        ps = plsc.cumsum(jnp.where(mask, 1, 0))
                        pos = nloc + ps - 1
                        for p in range(5):
                            v = cur[p][pl.ds(i * 64 + u * 16, 16)]
                            plsc.store_scatter(css[p], [pos], v, mask=mask)
                        nloc = nloc + plsc.all_reduce_population_count(mask)
                    return nloc

                lax.fori_loop(0, NVR // 4, kbody, jnp.zeros((16,), i32))
                padidx = n_c + iota
                plsc.store_scatter(cs0, [padidx], neg1, mask=padidx < n16c)
                nch = (n16c + 511) >> 9

                def dbody(kk, _):
                    hs = [pltpu.async_copy(
                        css[p].at[pl.ds(kk * 512, 512)],
                        spms[p].at[pl.ds(
                            pl.multiple_of(off_c + kk * 512, 512), 512)],
                        sem) for p in range(5)]
                    for hnd in hs:
                        hnd.wait()
                    return 0

                lax.fori_loop(0, nch, dbody, 0)

        plsc.subcore_barrier()

        @pl.when(sid == 0)
        def _tile0():
            def zbody(i, _):
                out_buf[pl.ds(i * 16, 16)] = zero16f
                return 0
            lax.fori_loop(0, 52, zbody, 0)

            m_j = [maxf_loc[pl.ds(j * 16, 16)] for j in range(6)]
            enc_j = [off_j[j] * 8192 + cnt16_j[j] for j in range(6)]

            def it(t, m_state):
                vm = m_state[0]
                for j in range(1, 6):
                    vm = jnp.maximum(vm, m_state[j])
                vstar = jnp.max(vm)
                cond = vstar > 0.0
                csel = jnp.full((16,), BIG, i32)
                encv = jnp.zeros((16,), i32)
                for j in range(6):
                    hit = m_state[j] == vstar
                    csel = jnp.minimum(csel,
                                       jnp.where(hit, iota + 16 * j, BIG))
                cstar = jnp.min(csel)
                for j in range(6):
                    at_c = (iota + 16 * j) == cstar
                    encv = encv + jnp.where(at_c, enc_j[j], 0)
                enc = jnp.max(encv)
                n16 = jnp.minimum(enc & 8191, P_PAD)
                off = pl.multiple_of(
                    jnp.minimum(enc >> 13, CAP - P_PAD), 512)
                nch = (n16 + 511) >> 9
                nv = n16 >> 4

                def fetch(kk, best):
                    offk = pl.multiple_of(off + kk * 512, 512)
                    hs = [pltpu.async_copy(
                        spms[p].at[pl.ds(offk, 512)],
                        rows[p].at[pl.ds(kk * 512, 512)], sem)
                        for p in range(5)]
                    hs[0].wait()
                    nvk = jnp.minimum(nv - kk * 32, 32)

                    def abody(i, b, kk=kk):
                        sv = row0[pl.ds(kk * 512 + i * 16, 16)]
                        return jnp.minimum(
                            b, jnp.where(sv == vstar,
                                         kk * 512 + i * 16 + iota, BIG))

                    best = lax.fori_loop(0, nvk, abody, best)
                    for hnd in hs[1:]:
                        hnd.wait()
                    return best

                bestv = lax.fori_loop(0, nch, fetch,
                                      jnp.full((16,), BIG, i32))
                irow = jnp.minimum(jnp.min(bestv), P_PAD - 16)

                irow_splat = jnp.full((16,), irow, i32)
                bx1 = plsc.load_gather(row1, [irow_splat])
                by1 = plsc.load_gather(row2, [irow_splat])
                bx2 = plsc.load_gather(row3, [irow_splat])
                by2 = plsc.load_gather(row4, [irow_splat])
                barea = (bx2 - bx1) * (by2 - by1)

                def ibody(i, runmax):
                    sl = pl.ds(i * 16, 16)
                    sv = row0[sl]
                    x1v = row1[sl]
                    y1v = row2[sl]
                    x2v = row3[sl]
                    y2v = row4[sl]
                    xx1 = jnp.maximum(bx1, x1v)
                    yy1 = jnp.maximum(by1, y1v)
                    xx2 = jnp.minimum(bx2, x2v)
                    yy2 = jnp.minimum(by2, y2v)
                    inter = (jnp.maximum(xx2 - xx1, 0.0)
                             * jnp.maximum(yy2 - yy1, 0.0))
                    areas = (x2v - x1v) * (y2v - y1v)
                    iou = inter / (barea + areas - inter)
                    supp = (iou > NMS_THRESH) | ((i * 16 + iota) == irow)
                    snew = jnp.where(supp, NEG, sv)
                    row0[sl] = snew
                    return jnp.maximum(runmax, snew)

                runmax = lax.fori_loop(0, nv, ibody, neg1)
                newmax = jnp.max(runmax)
                m_new = tuple(
                    jnp.where(((iota + 16 * j) == cstar) & cond, newmax,
                              m_state[j])
                    for j in range(6))

                payload = (jnp.where(iota == 0, bx1, 0.0)
                           + jnp.where(iota == 1, by1, 0.0)
                           + jnp.where(iota == 2, bx2, 0.0)
                           + jnp.where(iota == 3, by2, 0.0)
                           + jnp.where(iota == 4, vstar, 0.0)
                           + jnp.where(iota == 5, cstar.astype(f32), 0.0))
                plsc.store_scatter(out_buf, [t * 8 + iota], payload,
                                   mask=(iota < 8) & cond)

                def wbody(kk, _):
                    pltpu.sync_copy(
                        row0.at[pl.ds(kk * 512, 512)],
                        spm_s.at[pl.ds(pl.multiple_of(off + kk * 512, 512),
                                       512)])
                    return 0

                lax.fori_loop(0, nch, wbody, 0)
                return m_new

            lax.fori_loop(0, DETS_PER_IMG, it, tuple(m_j))
            pltpu.sync_copy(out_buf, out_hbm)


@functools.partial(jax.jit)
def kernel(class_logits, box_regression, proposals):
    f32 = jnp.float32
    lt = jnp.zeros((C_PAD, P_PAD), f32)
    lt = lt.at[:NUM_CLASSES, :N].set(class_logits.T)
    lt = lt.reshape(C_PAD, P_SUB, P_LANE)

    br = box_regression.reshape(N, NUM_CLASSES, 4)
    planes = []
    for k in range(4):
        pk = jnp.zeros((C_PAD, P_PAD), f32)
        pk = pk.at[:NUM_CLASSES, :N].set(br[:, :, k].T)
        planes.append(pk.reshape(C_PAD, P_SUB, P_LANE))
    dx_t, dy_t, dw_t, dh_t = planes

    prop = jnp.zeros((4, P_PAD), f32)
    prop = prop.at[:, :N].set(proposals.T)
    prop = prop.reshape(4, P_SUB, P_LANE)

    plane = jax.ShapeDtypeStruct((C_PAD, P_SUB, P_LANE), f32)
    vec = jax.ShapeDtypeStruct((8, 128), f32)
    s, x1, y1, x2, y2, cnts, maxes = pl.pallas_call(
        _decode_body,
        out_shape=[plane] * 5 + [vec, vec],
    )(lt, dx_t, dy_t, dw_t, dh_t, prop)
    cnts = cnts.reshape(1024)
    maxes = maxes.reshape(1024)

    s2 = s.reshape(C_PAD, P_PAD)
    x12 = x1.reshape(C_PAD, P_PAD)
    y12 = y1.reshape(C_PAD, P_PAD)
    x22 = x2.reshape(C_PAD, P_PAD)
    y22 = y2.reshape(C_PAD, P_PAD)

    mesh = plsc.VectorSubcoreMesh(core_axis_name="c", subcore_axis_name="s")
    sc_nms = pl.kernel(
        _sc_nms_body,
        out_type=jax.ShapeDtypeStruct((832,), f32),
        compiler_params=pltpu.CompilerParams(needs_layout_passes=False),
        mesh=mesh,
        scratch_types=[
            pltpu.VMEM((P_PAD,), f32),
            pltpu.VMEM((P_PAD,), f32),
            pltpu.VMEM((P_PAD,), f32),
            pltpu.VMEM((P_PAD,), f32),
            pltpu.VMEM((P_PAD,), f32),
            pltpu.VMEM((P_PAD,), f32),
            pltpu.VMEM((P_PAD,), f32),
            pltpu.VMEM((P_PAD,), f32),
            pltpu.VMEM((P_PAD,), f32),
            pltpu.VMEM((P_PAD,), f32),
            pltpu.VMEM((P_PAD,), f32),
            pltpu.VMEM((P_PAD,), f32),
            pltpu.VMEM((P_PAD,), f32),
            pltpu.VMEM((P_PAD,), f32),
            pltpu.VMEM((P_PAD,), f32),
            pltpu.VMEM((832,), f32),
            pltpu.VMEM((128,), f32),
            pltpu.VMEM((128,), f32),
            pltpu.VMEM_SHARED((CAP,), f32),
            pltpu.VMEM_SHARED((CAP,), f32),
            pltpu.VMEM_SHARED((CAP,), f32),
            pltpu.VMEM_SHARED((CAP,), f32),
            pltpu.VMEM_SHARED((CAP,), f32),
            pltpu.SemaphoreType.DMA,
        ],
    )
    packed = sc_nms(s2, x12, y12, x22, y22, cnts, maxes).reshape(104, 8)

    out_boxes = packed[:DETS_PER_IMG, 0:4]
    out_scores = packed[:DETS_PER_IMG, 4]
    out_labels = packed[:DETS_PER_IMG, 5].astype(jnp.int32)
    return out_boxes, out_scores, out_labels
